# Initial kernel scaffold; baseline (speedup 1.0000x reference)
import jax
import jax.numpy as jnp
from jax.experimental import pallas as pl


def kernel(x, embed, training):
    raise NotImplementedError("write your pallas kernel here")


# speedup vs baseline: 7.9022x; 7.9022x over previous
import jax
import jax.numpy as jnp
from jax.experimental import pallas as pl
from jax.experimental.pallas import tpu as pltpu

_K = 32


def _vq_body(x_ref, et_ref, e_ref, q_ref, loss_ref):
    b = pl.program_id(0)
    t = pl.program_id(1)
    xb = x_ref[0]
    et = et_ref[...]
    s = jnp.dot(et, xb, preferred_element_type=jnp.float32)
    e2 = jnp.sum(et * et, axis=1, keepdims=True)
    dist = e2 - 2.0 * s
    m = jnp.min(dist, axis=0, keepdims=True)
    kio = jax.lax.broadcasted_iota(jnp.int32, dist.shape, 0)
    idx = jnp.min(jnp.where(dist == m, kio, _K), axis=0, keepdims=True)
    oh = (kio == idx).astype(jnp.float32)
    q = jnp.dot(e_ref[...], oh, preferred_element_type=jnp.float32)
    q_ref[0] = q
    part = jnp.sum((q - xb) * (q - xb)).reshape(1, 1)

    @pl.when((b == 0) & (t == 0))
    def _init():
        loss_ref[...] = jnp.zeros((1, 1), jnp.float32)

    loss_ref[...] += part


def kernel(x, embed, training):
    B, C, T = x.shape
    Tb = 2048
    et = jnp.transpose(embed)
    grid = (B, T // Tb)
    q, loss_sum = pl.pallas_call(
        _vq_body,
        grid=grid,
        in_specs=[
            pl.BlockSpec((1, C, Tb), lambda b, t: (b, 0, t)),
            pl.BlockSpec((_K, C), lambda b, t: (0, 0)),
            pl.BlockSpec((C, _K), lambda b, t: (0, 0)),
        ],
        out_specs=[
            pl.BlockSpec((1, C, Tb), lambda b, t: (b, 0, t)),
            pl.BlockSpec((1, 1), lambda b, t: (0, 0)),
        ],
        out_shape=[
            jax.ShapeDtypeStruct((B, C, T), jnp.float32),
            jax.ShapeDtypeStruct((1, 1), jnp.float32),
        ],
    )(x, et, embed)
    loss = loss_sum[0, 0] / (B * C * T)
    return (q, embed, loss)


# speedup vs baseline: 10.7244x; 1.3571x over previous
import jax
import jax.numpy as jnp
from jax.experimental import pallas as pl
from jax.experimental.pallas import tpu as pltpu

_K = 32


def _vq_body(x_ref, et_ref, e_ref, q_ref, loss_ref):
    b = pl.program_id(0)
    t = pl.program_id(1)
    xb = x_ref[0]
    et = et_ref[...]
    s = jnp.dot(et, xb, preferred_element_type=jnp.float32)
    e2 = jnp.sum(et * et, axis=1, keepdims=True)
    dist = e2 - 2.0 * s
    m = jnp.min(dist, axis=0, keepdims=True)
    kio = jax.lax.broadcasted_iota(jnp.int32, dist.shape, 0)
    idx = jnp.min(jnp.where(dist == m, kio, _K), axis=0, keepdims=True)
    oh = (kio == idx).astype(jnp.float32)
    q = jnp.dot(e_ref[...], oh, preferred_element_type=jnp.float32)
    q_ref[0] = q
    part = (jnp.sum(xb * xb) + jnp.sum(m)).reshape(1, 1)

    @pl.when((b == 0) & (t == 0))
    def _init():
        loss_ref[...] = jnp.zeros((1, 1), jnp.float32)

    loss_ref[...] += part


def kernel(x, embed, training):
    B, C, T = x.shape
    Tb = 4096
    et = jnp.transpose(embed)
    grid = (B, T // Tb)
    q, loss_sum = pl.pallas_call(
        _vq_body,
        grid=grid,
        in_specs=[
            pl.BlockSpec((1, C, Tb), lambda b, t: (b, 0, t)),
            pl.BlockSpec((_K, C), lambda b, t: (0, 0)),
            pl.BlockSpec((C, _K), lambda b, t: (0, 0)),
        ],
        out_specs=[
            pl.BlockSpec((1, C, Tb), lambda b, t: (b, 0, t)),
            pl.BlockSpec((1, 1), lambda b, t: (0, 0)),
        ],
        out_shape=[
            jax.ShapeDtypeStruct((B, C, T), jnp.float32),
            jax.ShapeDtypeStruct((1, 1), jnp.float32),
        ],
    )(x, et, embed)
    loss = loss_sum[0, 0] / (B * C * T)
    return (q, embed, loss)


# speedup vs baseline: 11.2538x; 1.0494x over previous
import jax
import jax.numpy as jnp
from jax.experimental import pallas as pl
from jax.experimental.pallas import tpu as pltpu

_K = 32
_BB = 2


def _vq_body(x_ref, et_ref, e_ref, q_ref, loss_ref):
    g = pl.program_id(0)
    et = et_ref[...]
    e2 = jnp.sum(et * et, axis=1, keepdims=True)
    part = jnp.zeros((1, 1), jnp.float32)
    for i in range(_BB):
        xb = x_ref[i]
        s = jnp.dot(et, xb, preferred_element_type=jnp.float32)
        dist = e2 - 2.0 * s
        m = jnp.min(dist, axis=0, keepdims=True)
        kio = jax.lax.broadcasted_iota(jnp.int32, dist.shape, 0)
        idx = jnp.min(jnp.where(dist == m, kio, _K), axis=0, keepdims=True)
        oh = (kio == idx).astype(jnp.float32)
        q = jnp.dot(e_ref[...], oh, preferred_element_type=jnp.float32)
        q_ref[i] = q
        part = part + (jnp.sum(xb * xb) + jnp.sum(m)).reshape(1, 1)

    @pl.when(g == 0)
    def _init():
        loss_ref[...] = jnp.zeros((1, 1), jnp.float32)

    loss_ref[...] += part


def kernel(x, embed, training):
    B, C, T = x.shape
    et = jnp.transpose(embed)
    grid = (B // _BB,)
    q, loss_sum = pl.pallas_call(
        _vq_body,
        grid=grid,
        in_specs=[
            pl.BlockSpec((_BB, C, T), lambda g: (g, 0, 0)),
            pl.BlockSpec((_K, C), lambda g: (0, 0)),
            pl.BlockSpec((C, _K), lambda g: (0, 0)),
        ],
        out_specs=[
            pl.BlockSpec((_BB, C, T), lambda g: (g, 0, 0)),
            pl.BlockSpec((1, 1), lambda g: (0, 0)),
        ],
        out_shape=[
            jax.ShapeDtypeStruct((B, C, T), jnp.float32),
            jax.ShapeDtypeStruct((1, 1), jnp.float32),
        ],
    )(x, et, embed)
    loss = loss_sum[0, 0] / (B * C * T)
    return (q, embed, loss)
